# baseline (speedup 1.0000x reference)
import jax
import jax.numpy as jnp
from jax import lax
from jax.experimental import pallas as pl
from jax.experimental.pallas import tpu as pltpu
from jax.experimental.pallas import tpu_sc as plsc

NUM_EMB = 100000
EMB_DIM = 64
BATCH = 16384

_NPH = 8

_VOC = 99968
_TAIL = NUM_EMB - _VOC
_CH = 3200


def _detile_body(tab_hbm, lin_hbm, stage0, stage1, stage2, stage3,
                 w0, w1, w2, w3, r0, r1, r2, r3):
    w = lax.axis_index("s") * 2 + lax.axis_index("c")
    band = w >> 2
    q = w & 3
    stages = (stage0, stage1, stage2, stage3)
    wsems = (w0, w1, w2, w3)
    rsems = (r0, r1, r2, r3)

    def run_branch(m0, lengths):
        n = len(lengths)
        offs = []
        off = 0
        for length in lengths:
            offs.append(off)
            off += length

        def read_pair(j):
            return (
                tab_hbm.at[pl.ds(band * 8, 8), pl.ds(m0 + offs[j], lengths[j])],
                stages[j & 3].at[:, pl.ds(0, lengths[j])],
            )

        def writes(j):
            m = m0 + offs[j]
            return [
                (
                    stages[j & 3].at[i, pl.ds(0, lengths[j])],
                    lin_hbm.at[pl.ds((band * 8 + i) * _VOC + m, lengths[j])],
                )
                for i in range(8)
            ]

        all_ws = {}
        for j in range(min(4, n)):
            s, d = read_pair(j)
            pltpu.async_copy(s, d, rsems[j & 3])
        for j in range(n):
            s, d = read_pair(j)
            pltpu.make_async_copy(s, d, rsems[j & 3]).wait()
            ws = writes(j)
            all_ws[j] = ws
            for s, d in ws:
                pltpu.async_copy(s, d, wsems[j & 3])
            if j >= 3 and j + 1 < n:
                for s, d in all_ws.pop(j - 3):
                    pltpu.make_async_copy(s, d, wsems[(j - 3) & 3]).wait()
                s, d = read_pair(j + 1)
                pltpu.async_copy(s, d, rsems[(j + 1) & 3])
        for j, ws in sorted(all_ws.items()):
            for s, d in ws:
                pltpu.make_async_copy(s, d, wsems[j & 3]).wait()

    @pl.when(q == 0)
    def _():
        run_branch(0, [_CH] * 7 + [196 * 128 - 7 * _CH])

    @pl.when(q > 0)
    def _():
        run_branch(q * 24960 + 128, [_CH] * 7 + [195 * 128 - 7 * _CH])


def _body(idx_hbm, tab_hbm, tail_hbm, out_hbm, row_v,
          idx_v0, idx_v1, idx_v2, idx_v3, out_v0, out_v1,
          row_sem, isem0, isem1, isem2, isem3, out_sem0, out_sem1):
    wid = lax.axis_index("s") * 2 + lax.axis_index("c")
    idx_bufs = (idx_v0, idx_v1, idx_v2, idx_v3)
    idx_sems = (isem0, isem1, isem2, isem3)
    out_bufs = (out_v0, out_v1)
    out_sems = (out_sem0, out_sem1)
    pending = {0: [], 1: []}

    def idx_pair(ph):
        return (idx_hbm.at[pl.ds(32 * ph, 32)], idx_bufs[ph & 3])

    for fi, c in enumerate((wid, wid + 32)):
        row_parts = [
            (
                tab_hbm.at[c, pl.ds(k * (_VOC // 8), _VOC // 8)],
                row_v.at[pl.ds(k * (_VOC // 8), _VOC // 8)],
            )
            for k in range(8)
        ]
        row_parts.append(
            (tail_hbm.at[pl.ds(c * _TAIL, _TAIL)], row_v.at[pl.ds(_VOC, _TAIL)])
        )
        for s, d in row_parts:
            pltpu.async_copy(s, d, row_sem)
        for ph in range(3):
            s, d = idx_pair(ph)
            pltpu.async_copy(s, d, idx_sems[ph & 3])
        for s, d in row_parts:
            pltpu.make_async_copy(s, d, row_sem).wait()
        cr = c // 8
        ci = c % 8
        for ph in range(_NPH):
            b = ph & 1
            s, d = idx_pair(ph)
            pltpu.make_async_copy(s, d, idx_sems[ph & 3]).wait()
            if ph + 3 < _NPH:
                s, d = idx_pair(ph + 3)
                pltpu.async_copy(s, d, idx_sems[(ph + 3) & 3])
            for src, dst in pending[b]:
                pltpu.make_async_copy(src, dst, out_sems[b]).wait()
            pending[b] = []

            idx_buf = idx_bufs[ph & 3]
            out_buf = out_bufs[b]

            @plsc.parallel_loop(0, 256, unroll=8)
            def _(i):
                r = i >> 3
                tq = i >> 4
                p = (i >> 3) & 1
                g = (i & 7) << 4
                iv = idx_buf[r, pl.ds(g, 16)]
                out_buf[p, tq, pl.ds(g, 16)] = plsc.load_gather(row_v, [iv])

            for p in range(2):
                src = out_buf.at[p]
                dst = out_hbm.at[p, cr, pl.ds(16 * ph, 16), ci, :]
                pltpu.async_copy(src, dst, out_sems[b])
                pending[b].append((src, dst))

    for b in range(2):
        for src, dst in pending[b]:
            pltpu.make_async_copy(src, dst, out_sems[b]).wait()


@jax.jit
def kernel(labels, att_baseline):
    tab_t = att_baseline.T
    idx = labels.reshape(128, 128, 2).transpose(0, 2, 1).reshape(256, 128)
    mesh = plsc.VectorSubcoreMesh(core_axis_name="c", subcore_axis_name="s")
    lin = pl.kernel(
        _detile_body,
        out_type=jax.ShapeDtypeStruct((64 * _VOC,), jnp.float32),
        mesh=mesh,
        scratch_types=(
            [pltpu.VMEM((8, _CH), jnp.float32)] * 4
            + [pltpu.SemaphoreType.DMA] * 8
        ),
    )(tab_t)
    tab_lin = lin.reshape(64, _VOC)
    tail = tab_t[:, _VOC:].reshape(64 * _TAIL)
    a5 = pl.kernel(
        _body,
        out_type=jax.ShapeDtypeStruct((2, 8, 128, 8, 128), jnp.float32),
        mesh=mesh,
        scratch_types=[
            pltpu.VMEM((NUM_EMB,), jnp.float32),
            pltpu.VMEM((32, 128), jnp.int32),
            pltpu.VMEM((32, 128), jnp.int32),
            pltpu.VMEM((32, 128), jnp.int32),
            pltpu.VMEM((32, 128), jnp.int32),
            pltpu.VMEM((2, 16, 128), jnp.float32),
            pltpu.VMEM((2, 16, 128), jnp.float32),
            pltpu.SemaphoreType.DMA,
            pltpu.SemaphoreType.DMA,
            pltpu.SemaphoreType.DMA,
            pltpu.SemaphoreType.DMA,
            pltpu.SemaphoreType.DMA,
            pltpu.SemaphoreType.DMA,
            pltpu.SemaphoreType.DMA,
        ],
        compiler_params=pltpu.CompilerParams(
            use_tc_tiling_on_sc=False, needs_layout_passes=False
        ),
    )(idx, tab_lin, tail)
    return a5.transpose(2, 4, 0, 1, 3).reshape(BATCH, 2, EMB_DIM)


# speedup vs baseline: 1.0414x; 1.0414x over previous
import jax
import jax.numpy as jnp
from jax import lax
from jax.experimental import pallas as pl
from jax.experimental.pallas import tpu as pltpu
from jax.experimental.pallas import tpu_sc as plsc

NUM_EMB = 100000
EMB_DIM = 64
BATCH = 16384

_NPH = 8

_VOC = 99968
_TAIL = NUM_EMB - _VOC
_CH = 6272


def _detile_body(tab_hbm, lin_hbm, stage0, stage1, sem0, sem1, rsem0, rsem1):
    w = lax.axis_index("s") * 2 + lax.axis_index("c")
    band = w >> 3
    q = w & 7
    stages = (stage0, stage1)
    sems = (sem0, sem1)

    def run_branch(m0, lengths):
        n = len(lengths)
        offs = []
        off = 0
        for length in lengths:
            offs.append(off)
            off += length
        rsems = (rsem0, rsem1)

        def read_pair(j):
            return (
                tab_hbm.at[pl.ds(band * 8, 8), pl.ds(m0 + offs[j], lengths[j])],
                stages[j & 1].at[:, pl.ds(0, lengths[j])],
            )

        def writes(j):
            m = m0 + offs[j]
            b = j & 1
            return [
                (
                    stages[b].at[i, pl.ds(0, lengths[j])],
                    lin_hbm.at[pl.ds((band * 8 + i) * _VOC + m, lengths[j])],
                )
                for i in range(8)
            ]

        for j in (0, 1):
            s, d = read_pair(j)
            pltpu.async_copy(s, d, rsems[j & 1])
        tail_pending = []
        for j in range(n):
            b = j & 1
            s, d = read_pair(j)
            pltpu.make_async_copy(s, d, rsems[b]).wait()
            ws = writes(j)
            for s, d in ws:
                pltpu.async_copy(s, d, sems[b])
            if j + 2 < n:
                for s, d in ws:
                    pltpu.make_async_copy(s, d, sems[b]).wait()
                s, d = read_pair(j + 2)
                pltpu.async_copy(s, d, rsems[b])
            else:
                tail_pending.append((ws, b))
        for ws, b in tail_pending:
            for s, d in ws:
                pltpu.make_async_copy(s, d, sems[b]).wait()

    @pl.when(q < 5)
    def _():
        run_branch(q * (2 * _CH), [_CH, _CH])

    @pl.when(q >= 5)
    def _():
        run_branch(5 * (2 * _CH) + (q - 5) * (2 * _CH - 128), [_CH, _CH - 128])


def _body(idx_hbm, tab_hbm, tabhi_hbm, tail_hbm, out_hbm, row_v,
          idx_v0, idx_v1, idx_v2, idx_v3, out_v0, out_v1,
          row_sem, isem0, isem1, isem2, isem3, out_sem0, out_sem1):
    wid = lax.axis_index("s") * 2 + lax.axis_index("c")
    idx_bufs = (idx_v0, idx_v1, idx_v2, idx_v3)
    idx_sems = (isem0, isem1, isem2, isem3)
    out_bufs = (out_v0, out_v1)
    out_sems = (out_sem0, out_sem1)
    pending = {0: [], 1: []}

    def idx_pair(ph):
        return (idx_hbm.at[pl.ds(32 * ph, 32)], idx_bufs[ph & 3])

    for src_tab, c in ((tab_hbm, wid), (tabhi_hbm, wid + 32)):
        row_parts = [
            (
                src_tab.at[c - (0 if src_tab is tab_hbm else 32),
                           pl.ds(k * (_VOC // 4), _VOC // 4)],
                row_v.at[pl.ds(k * (_VOC // 4), _VOC // 4)],
            )
            for k in range(4)
        ]
        row_parts.append(
            (tail_hbm.at[pl.ds(c * _TAIL, _TAIL)], row_v.at[pl.ds(_VOC, _TAIL)])
        )
        for s, d in row_parts:
            pltpu.async_copy(s, d, row_sem)
        for ph in range(3):
            s, d = idx_pair(ph)
            pltpu.async_copy(s, d, idx_sems[ph & 3])
        for s, d in row_parts:
            pltpu.make_async_copy(s, d, row_sem).wait()
        cr = c // 8
        ci = c % 8
        for ph in range(_NPH):
            b = ph & 1
            s, d = idx_pair(ph)
            pltpu.make_async_copy(s, d, idx_sems[ph & 3]).wait()
            if ph + 3 < _NPH:
                s, d = idx_pair(ph + 3)
                pltpu.async_copy(s, d, idx_sems[(ph + 3) & 3])
            for src, dst in pending[b]:
                pltpu.make_async_copy(src, dst, out_sems[b]).wait()
            pending[b] = []

            idx_buf = idx_bufs[ph & 3]
            out_buf = out_bufs[b]

            @plsc.parallel_loop(0, 256, unroll=8)
            def _(i):
                r = i >> 3
                tq = i >> 4
                p = (i >> 3) & 1
                g = (i & 7) << 4
                iv = idx_buf[r, pl.ds(g, 16)]
                out_buf[p, tq, pl.ds(g, 16)] = plsc.load_gather(row_v, [iv])

            for p in range(2):
                src = out_buf.at[p]
                dst = out_hbm.at[p, cr, pl.ds(16 * ph, 16), ci, :]
                pltpu.async_copy(src, dst, out_sems[b])
                pending[b].append((src, dst))

    for b in range(2):
        for src, dst in pending[b]:
            pltpu.make_async_copy(src, dst, out_sems[b]).wait()


@jax.jit
def kernel(labels, att_baseline):
    tab_t = att_baseline.T
    idx = labels.reshape(128, 128, 2).transpose(0, 2, 1).reshape(256, 128)
    mesh = plsc.VectorSubcoreMesh(core_axis_name="c", subcore_axis_name="s")
    lin = pl.kernel(
        _detile_body,
        out_type=jax.ShapeDtypeStruct((32 * _VOC,), jnp.float32),
        mesh=mesh,
        scratch_types=(
            [pltpu.VMEM((8, _CH), jnp.float32)] * 2
            + [pltpu.SemaphoreType.DMA] * 4
        ),
    )(tab_t)
    tab_lin = lin.reshape(32, _VOC)
    tab_hi = tab_t[32:, :_VOC]
    tail = tab_t[:, _VOC:].reshape(64 * _TAIL)
    a5 = pl.kernel(
        _body,
        out_type=jax.ShapeDtypeStruct((2, 8, 128, 8, 128), jnp.float32),
        mesh=mesh,
        scratch_types=[
            pltpu.VMEM((NUM_EMB,), jnp.float32),
            pltpu.VMEM((32, 128), jnp.int32),
            pltpu.VMEM((32, 128), jnp.int32),
            pltpu.VMEM((32, 128), jnp.int32),
            pltpu.VMEM((32, 128), jnp.int32),
            pltpu.VMEM((2, 16, 128), jnp.float32),
            pltpu.VMEM((2, 16, 128), jnp.float32),
            pltpu.SemaphoreType.DMA,
            pltpu.SemaphoreType.DMA,
            pltpu.SemaphoreType.DMA,
            pltpu.SemaphoreType.DMA,
            pltpu.SemaphoreType.DMA,
            pltpu.SemaphoreType.DMA,
            pltpu.SemaphoreType.DMA,
        ],
        compiler_params=pltpu.CompilerParams(
            use_tc_tiling_on_sc=False, needs_layout_passes=False
        ),
    )(idx, tab_lin, tab_hi, tail)
    return a5.transpose(2, 4, 0, 1, 3).reshape(BATCH, 2, EMB_DIM)


# speedup vs baseline: 1.0794x; 1.0365x over previous
import jax
import jax.numpy as jnp
from jax import lax
from jax.experimental import pallas as pl
from jax.experimental.pallas import tpu as pltpu
from jax.experimental.pallas import tpu_sc as plsc

NUM_EMB = 100000
EMB_DIM = 64
BATCH = 16384

_NPH = 8

_VOC = 99968
_TAIL = NUM_EMB - _VOC
_CH = 6272


def _detile_body(tab_hbm, lin_hbm, stage0, stage1, sem0, sem1, rsem0, rsem1):
    w = lax.axis_index("s") * 2 + lax.axis_index("c")
    band = w >> 2
    q = w & 3
    stages = (stage0, stage1)
    sems = (sem0, sem1)

    def run_branch(m0, lengths):
        n = len(lengths)
        offs = []
        off = 0
        for length in lengths:
            offs.append(off)
            off += length
        rsems = (rsem0, rsem1)

        def read_pair(j):
            return (
                tab_hbm.at[pl.ds(band * 8, 8), pl.ds(m0 + offs[j], lengths[j])],
                stages[j & 1].at[:, pl.ds(0, lengths[j])],
            )

        def writes(j):
            m = m0 + offs[j]
            b = j & 1
            return [
                (
                    stages[b].at[i, pl.ds(0, lengths[j])],
                    lin_hbm.at[pl.ds((band * 8 + i) * _VOC + m, lengths[j])],
                )
                for i in range(8)
            ]

        for j in (0, 1):
            s, d = read_pair(j)
            pltpu.async_copy(s, d, rsems[j & 1])
        tail_pending = []
        for j in range(n):
            b = j & 1
            s, d = read_pair(j)
            pltpu.make_async_copy(s, d, rsems[b]).wait()
            ws = writes(j)
            for s, d in ws:
                pltpu.async_copy(s, d, sems[b])
            if j + 2 < n:
                for s, d in ws:
                    pltpu.make_async_copy(s, d, sems[b]).wait()
                s, d = read_pair(j + 2)
                pltpu.async_copy(s, d, rsems[b])
            else:
                tail_pending.append((ws, b))
        for ws, b in tail_pending:
            for s, d in ws:
                pltpu.make_async_copy(s, d, sems[b]).wait()

    @pl.when(q == 0)
    def _():
        run_branch(0, [_CH, _CH, _CH, _CH])

    @pl.when(q > 0)
    def _():
        run_branch(q * 24960 + 128, [_CH, _CH, _CH, 24960 - 3 * _CH])


def _body(idx_hbm, tab_hbm, tail_hbm, out_hbm, row_v,
          idx_v0, idx_v1, idx_v2, idx_v3, out_v0, out_v1,
          row_sem, isem0, isem1, isem2, isem3, out_sem0, out_sem1):
    wid = lax.axis_index("s") * 2 + lax.axis_index("c")
    idx_bufs = (idx_v0, idx_v1, idx_v2, idx_v3)
    idx_sems = (isem0, isem1, isem2, isem3)
    out_bufs = (out_v0, out_v1)
    out_sems = (out_sem0, out_sem1)
    pending = {0: [], 1: []}

    def idx_pair(ph):
        return (idx_hbm.at[pl.ds(32 * ph, 32)], idx_bufs[ph & 3])

    for c in (wid, wid + 32):
        row_parts = [
            (
                tab_hbm.at[c, pl.ds(k * (_VOC // 4), _VOC // 4)],
                row_v.at[pl.ds(k * (_VOC // 4), _VOC // 4)],
            )
            for k in range(4)
        ]
        row_parts.append(
            (tail_hbm.at[pl.ds(c * _TAIL, _TAIL)], row_v.at[pl.ds(_VOC, _TAIL)])
        )
        for s, d in row_parts:
            pltpu.async_copy(s, d, row_sem)
        for ph in range(3):
            s, d = idx_pair(ph)
            pltpu.async_copy(s, d, idx_sems[ph & 3])
        for s, d in row_parts:
            pltpu.make_async_copy(s, d, row_sem).wait()
        cr = c // 8
        ci = c % 8
        for ph in range(_NPH):
            b = ph & 1
            s, d = idx_pair(ph)
            pltpu.make_async_copy(s, d, idx_sems[ph & 3]).wait()
            if ph + 3 < _NPH:
                s, d = idx_pair(ph + 3)
                pltpu.async_copy(s, d, idx_sems[(ph + 3) & 3])
            for src, dst in pending[b]:
                pltpu.make_async_copy(src, dst, out_sems[b]).wait()
            pending[b] = []

            idx_buf = idx_bufs[ph & 3]
            out_buf = out_bufs[b]

            @plsc.parallel_loop(0, 256, unroll=8)
            def _(i):
                r = i >> 3
                tq = i >> 4
                p = (i >> 3) & 1
                g = (i & 7) << 4
                iv = idx_buf[r, pl.ds(g, 16)]
                out_buf[p, tq, pl.ds(g, 16)] = plsc.load_gather(row_v, [iv])

            for p in range(2):
                src = out_buf.at[p]
                dst = out_hbm.at[p, cr, pl.ds(16 * ph, 16), ci, :]
                pltpu.async_copy(src, dst, out_sems[b])
                pending[b].append((src, dst))

    for b in range(2):
        for src, dst in pending[b]:
            pltpu.make_async_copy(src, dst, out_sems[b]).wait()


@jax.jit
def kernel(labels, att_baseline):
    tab_t = att_baseline.T
    idx = labels.reshape(128, 128, 2).transpose(0, 2, 1).reshape(256, 128)
    mesh = plsc.VectorSubcoreMesh(core_axis_name="c", subcore_axis_name="s")
    lin = pl.kernel(
        _detile_body,
        out_type=jax.ShapeDtypeStruct((64 * _VOC,), jnp.float32),
        mesh=mesh,
        scratch_types=(
            [pltpu.VMEM((8, _CH), jnp.float32)] * 2
            + [pltpu.SemaphoreType.DMA] * 4
        ),
    )(tab_t)
    tab_lin = lin.reshape(64, _VOC)
    tail = tab_t[:, _VOC:].reshape(64 * _TAIL)
    a5 = pl.kernel(
        _body,
        out_type=jax.ShapeDtypeStruct((2, 8, 128, 8, 128), jnp.float32),
        mesh=mesh,
        scratch_types=[
            pltpu.VMEM((NUM_EMB,), jnp.float32),
            pltpu.VMEM((32, 128), jnp.int32),
            pltpu.VMEM((32, 128), jnp.int32),
            pltpu.VMEM((32, 128), jnp.int32),
            pltpu.VMEM((32, 128), jnp.int32),
            pltpu.VMEM((2, 16, 128), jnp.float32),
            pltpu.VMEM((2, 16, 128), jnp.float32),
            pltpu.SemaphoreType.DMA,
            pltpu.SemaphoreType.DMA,
            pltpu.SemaphoreType.DMA,
            pltpu.SemaphoreType.DMA,
            pltpu.SemaphoreType.DMA,
            pltpu.SemaphoreType.DMA,
            pltpu.SemaphoreType.DMA,
        ],
        compiler_params=pltpu.CompilerParams(
            use_tc_tiling_on_sc=False, needs_layout_passes=False
        ),
    )(idx, tab_lin, tail)
    return a5.transpose(2, 4, 0, 1, 3).reshape(BATCH, 2, EMB_DIM)
